# baseline (speedup 1.0000x reference)
import functools

import jax
import jax.numpy as jnp
from jax import lax
from jax.experimental import pallas as pl
from jax.experimental.pallas import tpu as pltpu
from jax.experimental.pallas import tpu_sc as plsc

_B = 1024
_A = 100
_ATTRS = 23
_EMB = 32
_FAN = _ATTRS * _EMB
_R = _B * _A
_R23 = _R * _ATTRS
_NC, _NS = 2, 16
_NW = _NC * _NS
_RW = _R // _NW
_STREAM = 128
_CROWS = 128
_CHUNK = _CROWS * _ATTRS
_NCHUNK = _RW // _CROWS

_mesh = functools.partial(plsc.VectorSubcoreMesh,
                          core_axis_name="c", subcore_axis_name="s")


def _wid():
    return lax.axis_index("s") * _NC + lax.axis_index("c")


@functools.partial(
    pl.kernel,
    mesh=_mesh(),
    compiler_params=pltpu.CompilerParams(use_tc_tiling_on_sc=False),
    out_type=jax.ShapeDtypeStruct((_R, _FAN), jnp.bfloat16),
    scratch_types=[
        pltpu.VMEM((_CHUNK,), jnp.int32),
        pltpu.VMEM((_CHUNK, _EMB), jnp.bfloat16),
        pltpu.SemaphoreType.DMA,
        pltpu.SemaphoreType.DMA,
    ],
)
def _sc_gather(idx_hbm, table_hbm, out_hbm, idx_v, rows_v, sem, sem_o):
    base = _wid() * _RW

    def body(ci, carry):
        row0 = base + ci * _CROWS
        pltpu.sync_copy(idx_hbm.at[pl.ds(row0 * _ATTRS, _CHUNK)], idx_v)
        copies = [
            pltpu.async_copy(
                table_hbm.at[idx_v.at[pl.ds(j * _STREAM, _STREAM)]],
                rows_v.at[pl.ds(j * _STREAM, _STREAM)],
                sem,
            )
            for j in range(_ATTRS)
        ]
        for c in copies:
            c.wait()
        outs = [
            pltpu.async_copy(
                rows_v.at[pl.ds(j * _STREAM, _STREAM)],
                out_hbm.at[pl.ds(row0, _CROWS), pl.ds(j * _EMB, _EMB)],
                sem_o,
            )
            for j in range(_ATTRS)
        ]
        for c in outs:
            c.wait()
        return carry

    lax.fori_loop(0, _NCHUNK, body, 0)


_BW = _B // _NW


@functools.partial(
    pl.kernel,
    mesh=_mesh(),
    compiler_params=pltpu.CompilerParams(use_tc_tiling_on_sc=False),
    out_type=jax.ShapeDtypeStruct((_B, _FAN), jnp.bfloat16),
    scratch_types=[
        pltpu.VMEM((_BW,), jnp.int32),
        pltpu.VMEM((_BW, _FAN), jnp.bfloat16),
        pltpu.SemaphoreType.DMA,
    ],
)
def _sc_my_gather(g_hbm, emb_hbm, out_hbm, g_v, rows_v, sem):
    base = _wid() * _BW
    pltpu.sync_copy(g_hbm.at[pl.ds(base, _BW)], g_v)
    pltpu.async_copy(emb_hbm.at[g_v], rows_v, sem).wait()
    pltpu.sync_copy(rows_v, out_hbm.at[pl.ds(base, _BW)])


def _rowsel_body(ids_ref, my_ref, g_ref):
    ids = ids_ref[...]
    match = (ids == my_ref[...]) & (ids != 0)
    lane = lax.broadcasted_iota(jnp.int32, ids.shape, 1)
    cand = jnp.where(match, lane, 16384)
    row = jnp.min(cand, axis=1, keepdims=True)
    row = jnp.where(row >= 16384, 0, row)
    bidx = lax.broadcasted_iota(jnp.int32, row.shape, 0)
    g_ref[...] = jnp.broadcast_to(bidx * _A + row, ids.shape)


def _rowsel(ids_pad, my2):
    return pl.pallas_call(
        _rowsel_body,
        out_shape=jax.ShapeDtypeStruct((_B, 128), jnp.int32),
    )(ids_pad, my2)


_MBLK = 1024


def _mm_body(x_ref, w_ref, b_ref, o_ref):
    o_ref[...] = (
        jnp.dot(x_ref[...], w_ref[...], preferred_element_type=jnp.float32)
        + b_ref[...]
    )


def _mm(emb, wt, bias):
    return pl.pallas_call(
        _mm_body,
        grid=(_R // _MBLK,),
        in_specs=[
            pl.BlockSpec((_MBLK, _FAN), lambda i: (i, 0)),
            pl.BlockSpec((_FAN, 512), lambda i: (0, 0)),
            pl.BlockSpec((1, 512), lambda i: (0, 0)),
        ],
        out_specs=pl.BlockSpec((_MBLK, 512), lambda i: (i, 0)),
        out_shape=jax.ShapeDtypeStruct((_R, 512), jnp.float32),
        compiler_params=pltpu.CompilerParams(
            dimension_semantics=("arbitrary",)
        ),
    )(emb, wt, bias)


def _myfc_body(x_ref, w_ref, b_ref, o_ref):
    o_ref[...] = jnp.maximum(
        jnp.dot(x_ref[...], w_ref[...], preferred_element_type=jnp.float32)
        + b_ref[...],
        0.0,
    )


def _myfc(x, wt, bias):
    return pl.pallas_call(
        _myfc_body,
        out_shape=jax.ShapeDtypeStruct((_B, 512), jnp.float32),
    )(x, wt, bias)


def kernel(agents, my_id, emb_table, agent_w, agent_b, my_w, my_b):
    idx = jnp.clip(agents, 0, 255) + jnp.arange(_ATTRS, dtype=jnp.int32) * 256
    idx_am = (
        idx.reshape(_R // _CROWS, _CROWS, _ATTRS)
        .transpose(0, 2, 1)
        .reshape(_R23)
    )
    emb = _sc_gather(idx_am, emb_table.astype(jnp.bfloat16))

    ids_pad = jnp.pad(agents[:, :, 0], ((0, 0), (0, 128 - _A)))
    my2 = jnp.broadcast_to(my_id[:, None], (_B, 128))
    g = _rowsel(ids_pad, my2)[:, 0]

    my_emb = _sc_my_gather(g, emb)

    agent_out = _mm(
        emb, agent_w.T.astype(jnp.bfloat16), agent_b[None, :]
    ).reshape(_B, _A, 512)
    my_out = _myfc(my_emb, my_w.T.astype(jnp.bfloat16), my_b[None, :])
    return agent_out, my_out


# speedup vs baseline: 1.0801x; 1.0801x over previous
import functools

import jax
import jax.numpy as jnp
from jax import lax
from jax.experimental import pallas as pl
from jax.experimental.pallas import tpu as pltpu
from jax.experimental.pallas import tpu_sc as plsc

_B = 1024
_A = 100
_AP = 104
_ATTRS = 23
_EMB = 32
_FAN = _ATTRS * _EMB
_R23 = _B * _A * _ATTRS
_RP = _B * _AP
_HID = 512
_NC, _NS = 2, 16
_NW = _NC * _NS
_CB = 2
_NCHUNK = _B // (_NW * _CB)
_AGC = _CB * _A * _ATTRS
_SLOT = _AP * _ATTRS
_SLOTP = 2400

_mesh = functools.partial(plsc.VectorSubcoreMesh,
                          core_axis_name="c", subcore_axis_name="s")


def _wid():
    return lax.axis_index("s") * _NC + lax.axis_index("c")


@functools.partial(
    pl.kernel,
    mesh=_mesh(),
    compiler_params=pltpu.CompilerParams(use_tc_tiling_on_sc=False, needs_layout_passes=False),
    out_type=jax.ShapeDtypeStruct((_RP, _FAN), jnp.bfloat16),
    scratch_types=[
        pltpu.VMEM((_CB * _SLOTP,), jnp.int32),
        pltpu.VMEM((_CB * _SLOTP,), jnp.int32),
        pltpu.VMEM((_AGC,), jnp.int32),
        pltpu.VMEM((_CB * _SLOTP,), jnp.int32),
        pltpu.VMEM((_SLOT, _EMB), jnp.bfloat16),
        pltpu.SemaphoreType.DMA,
        pltpu.SemaphoreType.DMA,
    ],
)
def _sc_gather(ag_hbm, posmap_hbm, offmap_hbm, table_hbm, out_hbm,
               pos_v, off_v, ag_v, idx_v, rows_v, sem_g, sem_o):
    pltpu.sync_copy(posmap_hbm, pos_v)
    pltpu.sync_copy(offmap_hbm, off_v)
    w = _wid()

    def chunk_body(ci, carry):
        gc = w * _NCHUNK + ci
        pltpu.sync_copy(ag_hbm.at[pl.ds(gc * _AGC, _AGC)], ag_v)

        def reorder(s, c2):
            s16 = s * 16
            pos = pos_v[pl.ds(s16, 16)]
            vals = plsc.load_gather(ag_v, [pos])
            vals = jnp.minimum(jnp.maximum(vals, 0), 255)
            idx_v[pl.ds(s16, 16)] = vals + off_v[pl.ds(s16, 16)]
            return c2

        lax.fori_loop(0, (_CB * _SLOTP) // 16, reorder, 0)

        def batch_body(q, c3):
            gb = gc * _CB + q
            gathers = [
                pltpu.async_copy(
                    table_hbm.at[
                        idx_v.at[pl.ds(q * _SLOTP + j * _AP, _AP)]
                    ],
                    rows_v.at[pl.ds(j * _AP, _AP)],
                    sem_g,
                )
                for j in range(_ATTRS)
            ]
            for c in gathers:
                c.wait()
            outs = [
                pltpu.async_copy(
                    rows_v.at[pl.ds(j * _AP, _AP)],
                    out_hbm.at[pl.ds(gb * _AP, _AP), pl.ds(j * _EMB, _EMB)],
                    sem_o,
                )
                for j in range(_ATTRS)
            ]
            for c in outs:
                c.wait()
            return c3

        lax.fori_loop(0, _CB, batch_body, 0)
        return carry

    lax.fori_loop(0, _NCHUNK, chunk_body, 0)


_BW = _B // _NW


@functools.partial(
    pl.kernel,
    mesh=_mesh(),
    compiler_params=pltpu.CompilerParams(use_tc_tiling_on_sc=False, needs_layout_passes=False),
    out_type=jax.ShapeDtypeStruct((_B, _FAN), jnp.bfloat16),
    scratch_types=[
        pltpu.VMEM((_BW,), jnp.int32),
        pltpu.VMEM((_BW, _FAN), jnp.bfloat16),
        pltpu.SemaphoreType.DMA,
    ],
)
def _sc_my_gather(g_hbm, emb_hbm, out_hbm, g_v, rows_v, sem):
    base = _wid() * _BW
    pltpu.sync_copy(g_hbm.at[pl.ds(base, _BW)], g_v)
    pltpu.async_copy(emb_hbm.at[g_v], rows_v, sem).wait()
    pltpu.sync_copy(rows_v, out_hbm.at[pl.ds(base, _BW)])


def _rowsel_body(ids_ref, my_ref, g_ref):
    ids = ids_ref[...]
    match = (ids == my_ref[...]) & (ids != 0)
    lane = lax.broadcasted_iota(jnp.int32, ids.shape, 1)
    cand = jnp.where(match, lane, 16384)
    row = jnp.min(cand, axis=1, keepdims=True)
    row = jnp.where(row >= 16384, 0, row)
    bidx = lax.broadcasted_iota(jnp.int32, row.shape, 0)
    g_ref[...] = jnp.broadcast_to(bidx * _AP + row, ids.shape)


def _rowsel(ids_pad, my2):
    return pl.pallas_call(
        _rowsel_body,
        out_shape=jax.ShapeDtypeStruct((_B, 128), jnp.int32),
    )(ids_pad, my2)


_BB = 8
_XB = _BB * _AP


def _mm_body(x_ref, w_ref, b_ref, o_ref):
    res = (
        jnp.dot(x_ref[...], w_ref[...], preferred_element_type=jnp.float32)
        + b_ref[...]
    )
    for i in range(_BB):
        o_ref[i] = res[i * _AP : i * _AP + _A]


def _mm(emb, wt, bias):
    return pl.pallas_call(
        _mm_body,
        grid=(_B // _BB,),
        in_specs=[
            pl.BlockSpec((_XB, _FAN), lambda i: (i, 0)),
            pl.BlockSpec((_FAN, _HID), lambda i: (0, 0)),
            pl.BlockSpec((1, _HID), lambda i: (0, 0)),
        ],
        out_specs=pl.BlockSpec((_BB, _A, _HID), lambda i: (i, 0, 0)),
        out_shape=jax.ShapeDtypeStruct((_B, _A, _HID), jnp.float32),
        compiler_params=pltpu.CompilerParams(
            dimension_semantics=("arbitrary",)
        ),
    )(emb, wt, bias)


def _myfc_body(x_ref, w_ref, b_ref, o_ref):
    o_ref[...] = jnp.maximum(
        jnp.dot(x_ref[...], w_ref[...], preferred_element_type=jnp.float32)
        + b_ref[...],
        0.0,
    )


def _myfc(x, wt, bias):
    return pl.pallas_call(
        _myfc_body,
        out_shape=jax.ShapeDtypeStruct((_B, _HID), jnp.float32),
    )(x, wt, bias)


def _make_maps():
    s = jnp.arange(_CB * _SLOTP, dtype=jnp.int32)
    q = s // _SLOTP
    t = s % _SLOTP
    j = t // _AP
    r = t % _AP
    valid = (j < _ATTRS) & (r < _A)
    posmap = jnp.where(valid, (r * _ATTRS + j) + q * (_A * _ATTRS), 0)
    offmap = jnp.where(j < _ATTRS, j * 256, 0)
    return posmap, offmap


def kernel(agents, my_id, emb_table, agent_w, agent_b, my_w, my_b):
    posmap, offmap = _make_maps()
    emb = _sc_gather(
        agents.reshape(_R23), posmap, offmap, emb_table.astype(jnp.bfloat16)
    )

    ids_pad = jnp.pad(agents[:, :, 0], ((0, 0), (0, 128 - _A)))
    my2 = jnp.broadcast_to(my_id[:, None], (_B, 128))
    g = _rowsel(ids_pad, my2)[:, 0]

    my_emb = _sc_my_gather(g, emb)

    agent_out = _mm(emb, agent_w.T.astype(jnp.bfloat16), agent_b[None, :])
    my_out = _myfc(my_emb, my_w.T.astype(jnp.bfloat16), my_b[None, :])
    return agent_out, my_out
